# baseline (speedup 1.0000x reference)
import functools

import jax
import jax.numpy as jnp
from jax import lax
from jax.experimental import pallas as pl
from jax.experimental.pallas import tpu as pltpu

B, E, V, D = 64, 100, 50, 768
G = 16
STEPS = B // G


def _diag_bce(z, mask_ref, lab_ref, bias, n, g):
    bg = lax.broadcasted_iota(jnp.int32, (G, n, B), 0)
    c = lax.broadcasted_iota(jnp.int32, (G, n, B), 2)
    zd = jnp.sum(jnp.where(c == g * G + bg, z, 0.0), axis=2)
    w = (zd + bias) * mask_ref[...]
    y = lab_ref[...].astype(jnp.float32)
    bce = jnp.maximum(w, 0.0) - w * y + jnp.log1p(jnp.exp(-jnp.abs(w)))
    return jnp.sum(bce, axis=(0, 1), keepdims=True)


def _fused_kernel(ent_ref, ev_ref, sr_ref, emask_ref, vmask_ref,
                  elab_ref, vlab_ref, wa_ref, we_ref, ba_ref, be_ref,
                  out_ref, uat_scr, uet_scr):
    g = pl.program_id(0)

    @pl.when(g == 0)
    def _init():
        sr = sr_ref[...]
        uat_scr[...] = lax.dot_general(
            wa_ref[0], sr, (((1,), (1,)), ((), ())),
            preferred_element_type=jnp.float32).astype(jnp.bfloat16)
        uet_scr[...] = lax.dot_general(
            we_ref[0], sr, (((1,), (1,)), ((), ())),
            preferred_element_type=jnp.float32).astype(jnp.bfloat16)
        out_ref[...] = jnp.zeros((1, 1), jnp.float32)

    za = lax.dot_general(ent_ref[...].astype(jnp.bfloat16), uat_scr[...],
                         (((2,), (0,)), ((), ())),
                         preferred_element_type=jnp.float32)
    zv = lax.dot_general(ev_ref[...].astype(jnp.bfloat16), uet_scr[...],
                         (((2,), (0,)), ((), ())),
                         preferred_element_type=jnp.float32)

    sa = _diag_bce(za, emask_ref, elab_ref, ba_ref[0], E, g)
    sv = _diag_bce(zv, vmask_ref, vlab_ref, be_ref[0], V, g)
    out_ref[...] += (0.5 / (B * E)) * sa + (0.5 / (B * V)) * sv


@functools.partial(jax.jit, static_argnames=())
def kernel(entity_mat, sr_vec, ev_mat, entity_mask, evidence_mask,
           entity_labels, evidence_labels, W_answer, b_answer,
           W_evidence, b_evidence):
    out = pl.pallas_call(
        _fused_kernel,
        grid=(STEPS,),
        in_specs=[
            pl.BlockSpec((G, E, D), lambda g: (g, 0, 0)),
            pl.BlockSpec((G, V, D), lambda g: (g, 0, 0)),
            pl.BlockSpec((B, D), lambda g: (0, 0)),
            pl.BlockSpec((G, E), lambda g: (g, 0)),
            pl.BlockSpec((G, V), lambda g: (g, 0)),
            pl.BlockSpec((G, E), lambda g: (g, 0)),
            pl.BlockSpec((G, V), lambda g: (g, 0)),
            pl.BlockSpec((1, D, D), lambda g: (0, 0, 0)),
            pl.BlockSpec((1, D, D), lambda g: (0, 0, 0)),
            pl.BlockSpec(memory_space=pltpu.SMEM),
            pl.BlockSpec(memory_space=pltpu.SMEM),
        ],
        out_specs=pl.BlockSpec((1, 1), lambda g: (0, 0)),
        out_shape=jax.ShapeDtypeStruct((1, 1), jnp.float32),
        scratch_shapes=[
            pltpu.VMEM((D, B), jnp.bfloat16),
            pltpu.VMEM((D, B), jnp.bfloat16),
        ],
    )(entity_mat, ev_mat, sr_vec, entity_mask, evidence_mask,
      entity_labels, evidence_labels, W_answer, W_evidence,
      b_answer, b_evidence)
    return out[0, 0]


# speedup vs baseline: 1.0123x; 1.0123x over previous
import functools

import jax
import jax.numpy as jnp
from jax import lax
from jax.experimental import pallas as pl
from jax.experimental.pallas import tpu as pltpu

B, E, V, D = 64, 100, 50, 768
G = 8
STEPS = B // G


def _diag_bce(z, mask_ref, lab_ref, bias, n, g):
    bg = lax.broadcasted_iota(jnp.int32, (G, n, B), 0)
    c = lax.broadcasted_iota(jnp.int32, (G, n, B), 2)
    zd = jnp.sum(jnp.where(c == g * G + bg, z, 0.0), axis=2)
    w = (zd + bias) * mask_ref[...]
    y = lab_ref[...].astype(jnp.float32)
    bce = jnp.maximum(w, 0.0) - w * y + jnp.log1p(jnp.exp(-jnp.abs(w)))
    return jnp.sum(bce, axis=(0, 1), keepdims=True)


def _fused_kernel(ent_ref, ev_ref, sr_ref, emask_ref, vmask_ref,
                  elab_ref, vlab_ref, wa_ref, we_ref, ba_ref, be_ref,
                  out_ref, uat_scr, uet_scr):
    g = pl.program_id(0)

    @pl.when(g == 0)
    def _init():
        sr = sr_ref[...]
        uat_scr[...] = lax.dot_general(
            wa_ref[0], sr, (((1,), (1,)), ((), ())),
            preferred_element_type=jnp.float32).astype(jnp.bfloat16)
        uet_scr[...] = lax.dot_general(
            we_ref[0], sr, (((1,), (1,)), ((), ())),
            preferred_element_type=jnp.float32).astype(jnp.bfloat16)
        out_ref[...] = jnp.zeros((1, 1), jnp.float32)

    za = lax.dot_general(ent_ref[...].astype(jnp.bfloat16), uat_scr[...],
                         (((2,), (0,)), ((), ())),
                         preferred_element_type=jnp.float32)
    zv = lax.dot_general(ev_ref[...].astype(jnp.bfloat16), uet_scr[...],
                         (((2,), (0,)), ((), ())),
                         preferred_element_type=jnp.float32)

    sa = _diag_bce(za, emask_ref, elab_ref, ba_ref[0], E, g)
    sv = _diag_bce(zv, vmask_ref, vlab_ref, be_ref[0], V, g)
    out_ref[...] += (0.5 / (B * E)) * sa + (0.5 / (B * V)) * sv


@functools.partial(jax.jit, static_argnames=())
def kernel(entity_mat, sr_vec, ev_mat, entity_mask, evidence_mask,
           entity_labels, evidence_labels, W_answer, b_answer,
           W_evidence, b_evidence):
    out = pl.pallas_call(
        _fused_kernel,
        grid=(STEPS,),
        in_specs=[
            pl.BlockSpec((G, E, D), lambda g: (g, 0, 0)),
            pl.BlockSpec((G, V, D), lambda g: (g, 0, 0)),
            pl.BlockSpec((B, D), lambda g: (0, 0)),
            pl.BlockSpec((G, E), lambda g: (g, 0)),
            pl.BlockSpec((G, V), lambda g: (g, 0)),
            pl.BlockSpec((G, E), lambda g: (g, 0)),
            pl.BlockSpec((G, V), lambda g: (g, 0)),
            pl.BlockSpec((1, D, D), lambda g: (0, 0, 0)),
            pl.BlockSpec((1, D, D), lambda g: (0, 0, 0)),
            pl.BlockSpec(memory_space=pltpu.SMEM),
            pl.BlockSpec(memory_space=pltpu.SMEM),
        ],
        out_specs=pl.BlockSpec((1, 1), lambda g: (0, 0)),
        out_shape=jax.ShapeDtypeStruct((1, 1), jnp.float32),
        scratch_shapes=[
            pltpu.VMEM((D, B), jnp.bfloat16),
            pltpu.VMEM((D, B), jnp.bfloat16),
        ],
    )(entity_mat, ev_mat, sr_vec, entity_mask, evidence_mask,
      entity_labels, evidence_labels, W_answer, W_evidence,
      b_answer, b_evidence)
    return out[0, 0]
